# baseline (speedup 1.0000x reference)
import numpy as np
import jax
import jax.numpy as jnp
from jax import lax
from jax.experimental import pallas as pl
from jax.experimental.pallas import tpu as pltpu
from jax.experimental.pallas import tpu_sc as plsc

_N = 10000
_E = 320000
_DIN = 128
_EDIM = 16
_H = 128
_EOUT = 256

_NC = 2
_NS = 16
_NW = _NC * _NS
_NP = 10240
_ROWS_PT = _NP // _NS
_EPW = _E // _NW
_BLK = 400
_NBLK = _EPW // _BLK
_TN = 640


def _make_perm():
    try:
        try:
            dev = jax.local_devices(backend="cpu")[0]
            with jax.default_device(dev):
                p = np.asarray(jax.random.permutation(jax.random.key(1), _E))
        except Exception:
            p = np.asarray(jax.random.permutation(jax.random.key(1), _E))
    except Exception:
        p = np.arange(_E)
    return p.astype(np.int32)


_PERM = _make_perm()


def _sc_mesh():
    return plsc.VectorSubcoreMesh(core_axis_name="c", subcore_axis_name="s",
                                  num_cores=_NC, num_subcores=_NS)


def _sc_scatter_body(efeat, src, dst, perm, zeros_h, ones_h,
                     spos_o, sneg_o, din_o, dout_o,
                     feat0, feat1, featp_v, ones_v, zeros_v,
                     src0, src1, dst0, dst1, perm0, perm1,
                     acc_pos, acc_neg, acc_din, acc_dout,
                     sem0, sem1, sem_g):
    cid = lax.axis_index("c")
    sid = lax.axis_index("s")
    wid = cid * _NS + sid

    pltpu.sync_copy(zeros_h, zeros_v)
    pltpu.sync_copy(ones_h, ones_v)

    row0 = sid * _ROWS_PT
    pltpu.sync_copy(zeros_v, acc_pos.at[pl.ds(row0, _ROWS_PT)])
    pltpu.sync_copy(zeros_v, acc_neg.at[pl.ds(row0, _ROWS_PT)])
    pltpu.sync_copy(zeros_v, acc_din.at[pl.ds(row0, _ROWS_PT)])
    pltpu.sync_copy(zeros_v, acc_dout.at[pl.ds(row0, _ROWS_PT)])
    plsc.subcore_barrier()

    def loads(b, fv, sv, dv, pv, sem):
        base = wid * _EPW + b * _BLK
        pltpu.async_copy(efeat.at[pl.ds(base, _BLK)], fv, sem)
        pltpu.async_copy(src.at[pl.ds(base, _BLK)], sv, sem)
        pltpu.async_copy(dst.at[pl.ds(base, _BLK)], dv, sem)
        pltpu.async_copy(perm.at[pl.ds(base, _BLK)], pv, sem)

    def wait_loads(b, fv, sv, dv, pv, sem):
        base = wid * _EPW + b * _BLK
        pltpu.make_async_copy(efeat.at[pl.ds(base, _BLK)], fv, sem).wait()
        pltpu.make_async_copy(src.at[pl.ds(base, _BLK)], sv, sem).wait()
        pltpu.make_async_copy(dst.at[pl.ds(base, _BLK)], dv, sem).wait()
        pltpu.make_async_copy(perm.at[pl.ds(base, _BLK)], pv, sem).wait()

    def consume(fv, sv, dv, pv):
        g = pltpu.async_copy(efeat.at[pv], featp_v, sem_g)
        pltpu.sync_copy(fv, acc_pos.at[dv], add=True)
        pltpu.sync_copy(ones_v, acc_din.at[dv], add=True)
        pltpu.sync_copy(ones_v, acc_dout.at[sv], add=True)
        g.wait()
        pltpu.sync_copy(featp_v, acc_neg.at[dv], add=True)

    loads(0, feat0, src0, dst0, perm0, sem0)

    def g_body(g, carry):
        b0 = g * 2
        wait_loads(b0, feat0, src0, dst0, perm0, sem0)
        loads(b0 + 1, feat1, src1, dst1, perm1, sem1)
        consume(feat0, src0, dst0, perm0)
        wait_loads(b0 + 1, feat1, src1, dst1, perm1, sem1)
        loads(b0 + 2, feat0, src0, dst0, perm0, sem0)
        consume(feat1, src1, dst1, perm1)
        return carry

    lax.fori_loop(0, (_NBLK - 1) // 2, g_body, 0)
    wait_loads(_NBLK - 1, feat0, src0, dst0, perm0, sem0)
    consume(feat0, src0, dst0, perm0)
    plsc.subcore_barrier()

    rows = pl.ds(row0, _ROWS_PT)
    pltpu.sync_copy(acc_pos.at[rows], spos_o.at[cid, rows])
    pltpu.sync_copy(acc_neg.at[rows], sneg_o.at[cid, rows])
    pltpu.sync_copy(acc_din.at[rows], din_o.at[cid, rows])
    pltpu.sync_copy(acc_dout.at[rows], dout_o.at[cid, rows])


def _run_sc_scatter(ef, src, dst, perm):
    acc = jax.ShapeDtypeStruct((_NC, _NP, _EDIM), jnp.float32)
    f = pl.kernel(
        _sc_scatter_body,
        out_type=(acc, acc, acc, acc),
        mesh=_sc_mesh(),
        compiler_params=pltpu.CompilerParams(use_tc_tiling_on_sc=False),
        scratch_types=[
            pltpu.VMEM((_BLK, _EDIM), jnp.float32),
            pltpu.VMEM((_BLK, _EDIM), jnp.float32),
            pltpu.VMEM((_BLK, _EDIM), jnp.float32),
            pltpu.VMEM((_BLK, _EDIM), jnp.float32),
            pltpu.VMEM((_ROWS_PT, _EDIM), jnp.float32),
            pltpu.VMEM((_BLK,), jnp.int32),
            pltpu.VMEM((_BLK,), jnp.int32),
            pltpu.VMEM((_BLK,), jnp.int32),
            pltpu.VMEM((_BLK,), jnp.int32),
            pltpu.VMEM((_BLK,), jnp.int32),
            pltpu.VMEM((_BLK,), jnp.int32),
            pltpu.VMEM_SHARED((_NP, _EDIM), jnp.float32),
            pltpu.VMEM_SHARED((_NP, _EDIM), jnp.float32),
            pltpu.VMEM_SHARED((_NP, _EDIM), jnp.float32),
            pltpu.VMEM_SHARED((_NP, _EDIM), jnp.float32),
            pltpu.SemaphoreType.DMA,
            pltpu.SemaphoreType.DMA,
            pltpu.SemaphoreType.DMA,
        ],
    )
    zeros_h = jnp.zeros((_ROWS_PT, _EDIM), jnp.float32)
    ones_h = jnp.ones((_BLK, _EDIM), jnp.float32)
    return f(ef, src, dst, perm, zeros_h, ones_h)


def _b_body(nf_ref, spos_ref, sneg_ref, din_ref, dout_ref,
            wnT_ref, weT_ref, b_ref, wew_ref, beb_ref, dw_ref,
            tab_ref, c_ref, hp_s, hn_s, acc_ref, u_s):
    p = pl.program_id(0)
    i = pl.program_id(1)
    off = pl.multiple_of(i * _TN, _TN)

    @pl.when(p == 0)
    def _phase0():
        din = din_ref[0, :, 0:1] + din_ref[1, :, 0:1]
        dout = dout_ref[0, :, 0:1] + dout_ref[1, :, 0:1]
        inv = 1.0 / jnp.maximum(din, 1.0)
        sp = (spos_ref[0] + spos_ref[1]) * inv
        sn = (sneg_ref[0] + sneg_ref[1]) * inv
        base = jnp.dot(nf_ref[...], wnT_ref[...],
                       preferred_element_type=jnp.float32) + b_ref[...]
        weT = weT_ref[...]
        hp = jnp.maximum(
            base + jnp.dot(sp, weT, preferred_element_type=jnp.float32), 0.0)
        hn = jnp.maximum(
            base + jnp.dot(sn, weT, preferred_element_type=jnp.float32), 0.0)
        hp_s[pl.ds(off, _TN), :] = hp
        hn_s[pl.ds(off, _TN), :] = hn

        mask = ((lax.broadcasted_iota(jnp.int32, (_TN, 1), 0) + i * _TN)
                < _N).astype(jnp.float32)
        msrc_t = jnp.sum(hp * (dout * mask), axis=0, keepdims=True)
        mdst_t = jnp.sum(hp * (din * mask), axis=0, keepdims=True)

        @pl.when(i == 0)
        def _init():
            acc_ref[...] = jnp.zeros((2, _H), jnp.float32)

        acc_ref[0:1, :] += msrc_t
        acc_ref[1:2, :] += mdst_t

        @pl.when(i == (_NP // _TN) - 1)
        def _fin():
            msrc = acc_ref[0:1, :]
            mdst = acc_ref[1:2, :]
            wew = wew_ref[...]
            w1 = wew[:, :_H]
            w2 = wew[:, _H:]
            dims = (((1,), (1,)), ((), ()))
            me = (lax.dot_general(msrc, w1, dims,
                                  preferred_element_type=jnp.float32)
                  + lax.dot_general(mdst, w2, dims,
                                    preferred_element_type=jnp.float32)
                  ) * (1.0 / _E) + beb_ref[...]
            summ = jax.nn.sigmoid(me)
            ws = lax.dot_general(summ, dw_ref[...], dims,
                                 preferred_element_type=jnp.float32)
            dims2 = (((1,), (0,)), ((), ()))
            u1 = lax.dot_general(ws, w1, dims2,
                                 preferred_element_type=jnp.float32)
            u2 = lax.dot_general(ws, w2, dims2,
                                 preferred_element_type=jnp.float32)
            u_s[...] = jnp.concatenate([u1, u2], axis=0)
            c_ref[...] = jnp.sum(beb_ref[...] * ws).reshape(1, 1)

    @pl.when(p == 1)
    def _phase1():
        u = u_s[...]
        hp = hp_s[pl.ds(off, _TN), :]
        hn = hn_s[pl.ds(off, _TN), :]
        dims = (((1,), (1,)), ((), ()))
        pq_p = lax.dot_general(u, hp, dims,
                               preferred_element_type=jnp.float32)
        pq_n = lax.dot_general(u, hn, dims,
                               preferred_element_type=jnp.float32)
        tab_ref[...] = jnp.concatenate(
            [pq_p, pq_n, jnp.zeros((4, _TN), jnp.float32)], axis=0)


def _run_b(nf, spos, sneg, din, dout, wnT, weT, brow, wew, beb_row, dw):
    grid = (2, _NP // _TN)
    seg = pl.BlockSpec((2, _TN, _EDIM), lambda p, i: (0, i, 0))
    return pl.pallas_call(
        _b_body,
        grid=grid,
        in_specs=[
            pl.BlockSpec((_TN, _DIN), lambda p, i: (i, 0)),
            seg, seg, seg, seg,
            pl.BlockSpec((_DIN, _H), lambda p, i: (0, 0)),
            pl.BlockSpec((_EDIM, _H), lambda p, i: (0, 0)),
            pl.BlockSpec((1, _H), lambda p, i: (0, 0)),
            pl.BlockSpec((_EOUT, _EOUT), lambda p, i: (0, 0)),
            pl.BlockSpec((1, _EOUT), lambda p, i: (0, 0)),
            pl.BlockSpec((_EOUT, _EOUT), lambda p, i: (0, 0)),
        ],
        out_specs=[pl.BlockSpec((8, _TN), lambda p, i: (p, i)),
                   pl.BlockSpec((1, 1), lambda p, i: (0, 0))],
        out_shape=[jax.ShapeDtypeStruct((16, _NP), jnp.float32),
                   jax.ShapeDtypeStruct((1, 1), jnp.float32)],
        scratch_shapes=[pltpu.VMEM((_NP, _H), jnp.float32),
                        pltpu.VMEM((_NP, _H), jnp.float32),
                        pltpu.VMEM((2, _H), jnp.float32),
                        pltpu.VMEM((2, _H), jnp.float32)],
        compiler_params=pltpu.CompilerParams(
            vmem_limit_bytes=100 * 1024 * 1024),
    )(nf, spos, sneg, din, dout, wnT, weT, brow, wew, beb_row, dw)


def _sc_edge_body(src, dst, tab, xp_o, xn_o,
                  tab_v, src_v, dst_v, xp_v, xn_v):
    cid = lax.axis_index("c")
    sid = lax.axis_index("s")
    wid = cid * _NS + sid

    pltpu.sync_copy(tab, tab_v)
    base = wid * _EPW
    pltpu.sync_copy(src.at[pl.ds(base, _EPW)], src_v)
    pltpu.sync_copy(dst.at[pl.ds(base, _EPW)], dst_v)

    def it(i, carry):
        s = pl.ds(i * 16, 16)
        sv = src_v[s]
        dv = dst_v[s]
        xp_v[s] = (plsc.load_gather(tab_v, [sv])
                   + plsc.load_gather(tab_v, [dv + _NP]))
        xn_v[s] = (plsc.load_gather(tab_v, [sv + 2 * _NP])
                   + plsc.load_gather(tab_v, [dv + 3 * _NP]))
        return carry

    lax.fori_loop(0, _EPW // 16, it, 0)
    pltpu.sync_copy(xp_v, xp_o.at[pl.ds(base, _EPW)])
    pltpu.sync_copy(xn_v, xn_o.at[pl.ds(base, _EPW)])


def _run_sc_edge(src, dst, tab):
    out = jax.ShapeDtypeStruct((_E,), jnp.float32)
    f = pl.kernel(
        _sc_edge_body,
        out_type=(out, out),
        mesh=_sc_mesh(),
        compiler_params=pltpu.CompilerParams(use_tc_tiling_on_sc=False,
                                             needs_layout_passes=False),
        scratch_types=[
            pltpu.VMEM((4 * _NP,), jnp.float32),
            pltpu.VMEM((_EPW,), jnp.int32),
            pltpu.VMEM((_EPW,), jnp.int32),
            pltpu.VMEM((_EPW,), jnp.float32),
            pltpu.VMEM((_EPW,), jnp.float32),
        ],
    )
    return f(src, dst, tab)


def _d_body(xp_ref, xn_ref, c_ref, o_ref):
    c = c_ref[0, 0]
    lp = jnp.mean(jax.nn.softplus(-(xp_ref[...] + c)))
    ln = jnp.mean(jax.nn.softplus(xn_ref[...] + c))
    o_ref[...] = (lp + ln).reshape(1, 1)


def _run_d(xp2, xn2, c):
    return pl.pallas_call(
        _d_body,
        out_shape=jax.ShapeDtypeStruct((1, 1), jnp.float32),
    )(xp2, xn2, c)


def kernel(n_features, e_features, edge_index, W_apply_w, W_apply_b,
           W_edge_w, W_edge_b, disc_W):
    nf = jnp.concatenate(
        [n_features.reshape(_N, _DIN),
         jnp.zeros((_NP - _N, _DIN), jnp.float32)], axis=0)
    ef = e_features.reshape(_E, _EDIM)
    src = edge_index[0]
    dst = edge_index[1]
    perm = jnp.asarray(_PERM)

    spos, sneg, din, dout = _run_sc_scatter(ef, src, dst, perm)

    wnT = W_apply_w[:, :_DIN].T
    weT = W_apply_w[:, _DIN:].T
    brow = W_apply_b.reshape(1, _H)
    beb_row = W_edge_b.reshape(1, _EOUT)
    tab8, c = _run_b(nf, spos, sneg, din, dout,
                     wnT, weT, brow, W_edge_w, beb_row, disc_W)

    xp, xn = _run_sc_edge(src, dst, tab8[8:12].reshape(4 * _NP))

    loss = _run_d(xp.reshape(_E // _DIN, _DIN), xn.reshape(_E // _DIN, _DIN),
                  c)
    return loss[0, 0]


# speedup vs baseline: 1.0395x; 1.0395x over previous
import numpy as np
import jax
import jax.numpy as jnp
from jax import lax
from jax.experimental import pallas as pl
from jax.experimental.pallas import tpu as pltpu
from jax.experimental.pallas import tpu_sc as plsc

_N = 10000
_E = 320000
_DIN = 128
_EDIM = 16
_H = 128
_EOUT = 256

_NC = 2
_NS = 16
_NW = _NC * _NS
_NP = 10240
_ROWS_PT = _NP // _NS
_EPW = _E // _NW
_BLK = 400
_NBLK = _EPW // _BLK
_TN = 640


def _make_perm():
    try:
        try:
            dev = jax.local_devices(backend="cpu")[0]
            with jax.default_device(dev):
                p = np.asarray(jax.random.permutation(jax.random.key(1), _E))
        except Exception:
            p = np.asarray(jax.random.permutation(jax.random.key(1), _E))
    except Exception:
        p = np.arange(_E)
    return p.astype(np.int32)


_PERM = _make_perm()


def _sc_mesh():
    return plsc.VectorSubcoreMesh(core_axis_name="c", subcore_axis_name="s",
                                  num_cores=_NC, num_subcores=_NS)


def _sc_scatter_body(efeat, src, dst, perm, zeros_h, ones_h,
                     spos_o, sneg_o, din_o, dout_o,
                     feat0, feat1, featp_v, ones_v, zeros_v,
                     src0, src1, dst0, dst1, perm0, perm1,
                     acc_pos, acc_neg, acc_din, acc_dout,
                     sem0, sem1, sem_g):
    cid = lax.axis_index("c")
    sid = lax.axis_index("s")
    wid = cid * _NS + sid

    pltpu.sync_copy(zeros_h, zeros_v)
    pltpu.sync_copy(ones_h, ones_v)

    row0 = sid * _ROWS_PT
    pltpu.sync_copy(zeros_v, acc_pos.at[pl.ds(row0, _ROWS_PT)])
    pltpu.sync_copy(zeros_v, acc_neg.at[pl.ds(row0, _ROWS_PT)])
    pltpu.sync_copy(zeros_v, acc_din.at[pl.ds(row0, _ROWS_PT)])
    pltpu.sync_copy(zeros_v, acc_dout.at[pl.ds(row0, _ROWS_PT)])
    plsc.subcore_barrier()

    def loads(b, fv, sv, dv, pv, sem):
        base = wid * _EPW + b * _BLK
        pltpu.async_copy(efeat.at[pl.ds(base, _BLK)], fv, sem)
        pltpu.async_copy(src.at[pl.ds(base, _BLK)], sv, sem)
        pltpu.async_copy(dst.at[pl.ds(base, _BLK)], dv, sem)
        pltpu.async_copy(perm.at[pl.ds(base, _BLK)], pv, sem)

    def wait_loads(b, fv, sv, dv, pv, sem):
        base = wid * _EPW + b * _BLK
        pltpu.make_async_copy(efeat.at[pl.ds(base, _BLK)], fv, sem).wait()
        pltpu.make_async_copy(src.at[pl.ds(base, _BLK)], sv, sem).wait()
        pltpu.make_async_copy(dst.at[pl.ds(base, _BLK)], dv, sem).wait()
        pltpu.make_async_copy(perm.at[pl.ds(base, _BLK)], pv, sem).wait()

    def consume(fv, sv, dv, pv):
        g = pltpu.async_copy(efeat.at[pv], featp_v, sem_g)
        pltpu.sync_copy(fv, acc_pos.at[dv], add=True)
        pltpu.sync_copy(ones_v, acc_din.at[dv], add=True)
        pltpu.sync_copy(ones_v, acc_dout.at[sv], add=True)
        g.wait()
        pltpu.sync_copy(featp_v, acc_neg.at[dv], add=True)

    loads(0, feat0, src0, dst0, perm0, sem0)

    def g_body(g, carry):
        b0 = g * 2
        wait_loads(b0, feat0, src0, dst0, perm0, sem0)
        loads(b0 + 1, feat1, src1, dst1, perm1, sem1)
        consume(feat0, src0, dst0, perm0)
        wait_loads(b0 + 1, feat1, src1, dst1, perm1, sem1)
        loads(b0 + 2, feat0, src0, dst0, perm0, sem0)
        consume(feat1, src1, dst1, perm1)
        return carry

    lax.fori_loop(0, (_NBLK - 1) // 2, g_body, 0)
    wait_loads(_NBLK - 1, feat0, src0, dst0, perm0, sem0)
    consume(feat0, src0, dst0, perm0)
    plsc.subcore_barrier()

    rows = pl.ds(row0, _ROWS_PT)
    pltpu.sync_copy(acc_pos.at[rows], spos_o.at[cid, rows])
    pltpu.sync_copy(acc_neg.at[rows], sneg_o.at[cid, rows])
    pltpu.sync_copy(acc_din.at[rows], din_o.at[cid, rows])
    pltpu.sync_copy(acc_dout.at[rows], dout_o.at[cid, rows])


def _run_sc_scatter(ef, src, dst, perm):
    acc = jax.ShapeDtypeStruct((_NC, _NP, _EDIM), jnp.float32)
    f = pl.kernel(
        _sc_scatter_body,
        out_type=(acc, acc, acc, acc),
        mesh=_sc_mesh(),
        compiler_params=pltpu.CompilerParams(use_tc_tiling_on_sc=False),
        scratch_types=[
            pltpu.VMEM((_BLK, _EDIM), jnp.float32),
            pltpu.VMEM((_BLK, _EDIM), jnp.float32),
            pltpu.VMEM((_BLK, _EDIM), jnp.float32),
            pltpu.VMEM((_BLK, _EDIM), jnp.float32),
            pltpu.VMEM((_ROWS_PT, _EDIM), jnp.float32),
            pltpu.VMEM((_BLK,), jnp.int32),
            pltpu.VMEM((_BLK,), jnp.int32),
            pltpu.VMEM((_BLK,), jnp.int32),
            pltpu.VMEM((_BLK,), jnp.int32),
            pltpu.VMEM((_BLK,), jnp.int32),
            pltpu.VMEM((_BLK,), jnp.int32),
            pltpu.VMEM_SHARED((_NP, _EDIM), jnp.float32),
            pltpu.VMEM_SHARED((_NP, _EDIM), jnp.float32),
            pltpu.VMEM_SHARED((_NP, _EDIM), jnp.float32),
            pltpu.VMEM_SHARED((_NP, _EDIM), jnp.float32),
            pltpu.SemaphoreType.DMA,
            pltpu.SemaphoreType.DMA,
            pltpu.SemaphoreType.DMA,
        ],
    )
    zeros_h = jnp.zeros((_ROWS_PT, _EDIM), jnp.float32)
    ones_h = jnp.ones((_BLK, _EDIM), jnp.float32)
    return f(ef, src, dst, perm, zeros_h, ones_h)


def _b_body(nf_ref, spos_ref, sneg_ref, din_ref, dout_ref,
            wnT_ref, weT_ref, b_ref, wew_ref, beb_ref, dw_ref,
            tab_ref, c_ref, hp_s, hn_s, acc_ref, u_s):
    p = pl.program_id(0)
    i = pl.program_id(1)
    off = pl.multiple_of(i * _TN, _TN)

    @pl.when(p == 0)
    def _phase0():
        din = din_ref[0, :, 0:1] + din_ref[1, :, 0:1]
        dout = dout_ref[0, :, 0:1] + dout_ref[1, :, 0:1]
        inv = 1.0 / jnp.maximum(din, 1.0)
        sp = (spos_ref[0] + spos_ref[1]) * inv
        sn = (sneg_ref[0] + sneg_ref[1]) * inv
        base = jnp.dot(nf_ref[...], wnT_ref[...],
                       preferred_element_type=jnp.float32) + b_ref[...]
        weT = weT_ref[...]
        hp = jnp.maximum(
            base + jnp.dot(sp, weT, preferred_element_type=jnp.float32), 0.0)
        hn = jnp.maximum(
            base + jnp.dot(sn, weT, preferred_element_type=jnp.float32), 0.0)
        hp_s[pl.ds(off, _TN), :] = hp
        hn_s[pl.ds(off, _TN), :] = hn

        mask = ((lax.broadcasted_iota(jnp.int32, (_TN, 1), 0) + i * _TN)
                < _N).astype(jnp.float32)
        msrc_t = jnp.sum(hp * (dout * mask), axis=0, keepdims=True)
        mdst_t = jnp.sum(hp * (din * mask), axis=0, keepdims=True)

        @pl.when(i == 0)
        def _init():
            acc_ref[...] = jnp.zeros((2, _H), jnp.float32)

        acc_ref[0:1, :] += msrc_t
        acc_ref[1:2, :] += mdst_t

        @pl.when(i == (_NP // _TN) - 1)
        def _fin():
            msrc = acc_ref[0:1, :]
            mdst = acc_ref[1:2, :]
            wew = wew_ref[...]
            w1 = wew[:, :_H]
            w2 = wew[:, _H:]
            dims = (((1,), (1,)), ((), ()))
            me = (lax.dot_general(msrc, w1, dims,
                                  preferred_element_type=jnp.float32)
                  + lax.dot_general(mdst, w2, dims,
                                    preferred_element_type=jnp.float32)
                  ) * (1.0 / _E) + beb_ref[...]
            summ = jax.nn.sigmoid(me)
            ws = lax.dot_general(summ, dw_ref[...], dims,
                                 preferred_element_type=jnp.float32)
            dims2 = (((1,), (0,)), ((), ()))
            u1 = lax.dot_general(ws, w1, dims2,
                                 preferred_element_type=jnp.float32)
            u2 = lax.dot_general(ws, w2, dims2,
                                 preferred_element_type=jnp.float32)
            u_s[...] = jnp.concatenate([u1, u2], axis=0)
            c_ref[...] = jnp.sum(beb_ref[...] * ws).reshape(1, 1)

    @pl.when(p == 1)
    def _phase1():
        u = u_s[...]
        hp = hp_s[pl.ds(off, _TN), :]
        hn = hn_s[pl.ds(off, _TN), :]
        dims = (((1,), (1,)), ((), ()))
        pq_p = lax.dot_general(u, hp, dims,
                               preferred_element_type=jnp.float32)
        pq_n = lax.dot_general(u, hn, dims,
                               preferred_element_type=jnp.float32)
        tab_ref[...] = jnp.concatenate(
            [pq_p, pq_n, jnp.zeros((4, _TN), jnp.float32)], axis=0)


def _run_b(nf, spos, sneg, din, dout, wnT, weT, brow, wew, beb_row, dw):
    grid = (2, _NP // _TN)

    def _i0(p, i):
        return jnp.where(p == 0, i, 0)

    seg = pl.BlockSpec((2, _TN, _EDIM), lambda p, i: (0, _i0(p, i), 0))
    return pl.pallas_call(
        _b_body,
        grid=grid,
        in_specs=[
            pl.BlockSpec((_TN, _DIN), lambda p, i: (_i0(p, i), 0)),
            seg, seg, seg, seg,
            pl.BlockSpec((_DIN, _H), lambda p, i: (0, 0)),
            pl.BlockSpec((_EDIM, _H), lambda p, i: (0, 0)),
            pl.BlockSpec((1, _H), lambda p, i: (0, 0)),
            pl.BlockSpec((_EOUT, _EOUT), lambda p, i: (0, 0)),
            pl.BlockSpec((1, _EOUT), lambda p, i: (0, 0)),
            pl.BlockSpec((_EOUT, _EOUT), lambda p, i: (0, 0)),
        ],
        out_specs=[pl.BlockSpec((8, _TN), lambda p, i: (p, i)),
                   pl.BlockSpec((1, 1), lambda p, i: (0, 0))],
        out_shape=[jax.ShapeDtypeStruct((16, _NP), jnp.float32),
                   jax.ShapeDtypeStruct((1, 1), jnp.float32)],
        scratch_shapes=[pltpu.VMEM((_NP, _H), jnp.float32),
                        pltpu.VMEM((_NP, _H), jnp.float32),
                        pltpu.VMEM((2, _H), jnp.float32),
                        pltpu.VMEM((2, _H), jnp.float32)],
        compiler_params=pltpu.CompilerParams(
            vmem_limit_bytes=100 * 1024 * 1024),
    )(nf, spos, sneg, din, dout, wnT, weT, brow, wew, beb_row, dw)


def _sc_edge_body(src, dst, tab, xp_o, xn_o,
                  tab_v, src_v, dst_v, xp_v, xn_v):
    cid = lax.axis_index("c")
    sid = lax.axis_index("s")
    wid = cid * _NS + sid

    pltpu.sync_copy(tab, tab_v)
    base = wid * _EPW
    pltpu.sync_copy(src.at[pl.ds(base, _EPW)], src_v)
    pltpu.sync_copy(dst.at[pl.ds(base, _EPW)], dst_v)

    def it(i, carry):
        s = pl.ds(i * 16, 16)
        sv = src_v[s]
        dv = dst_v[s]
        xp_v[s] = (plsc.load_gather(tab_v, [sv])
                   + plsc.load_gather(tab_v, [dv + _NP]))
        xn_v[s] = (plsc.load_gather(tab_v, [sv + 2 * _NP])
                   + plsc.load_gather(tab_v, [dv + 3 * _NP]))
        return carry

    lax.fori_loop(0, _EPW // 16, it, 0)
    pltpu.sync_copy(xp_v, xp_o.at[pl.ds(base, _EPW)])
    pltpu.sync_copy(xn_v, xn_o.at[pl.ds(base, _EPW)])


def _run_sc_edge(src, dst, tab):
    out = jax.ShapeDtypeStruct((_E,), jnp.float32)
    f = pl.kernel(
        _sc_edge_body,
        out_type=(out, out),
        mesh=_sc_mesh(),
        compiler_params=pltpu.CompilerParams(use_tc_tiling_on_sc=False,
                                             needs_layout_passes=False),
        scratch_types=[
            pltpu.VMEM((4 * _NP,), jnp.float32),
            pltpu.VMEM((_EPW,), jnp.int32),
            pltpu.VMEM((_EPW,), jnp.int32),
            pltpu.VMEM((_EPW,), jnp.float32),
            pltpu.VMEM((_EPW,), jnp.float32),
        ],
    )
    return f(src, dst, tab)


def _d_body(xp_ref, xn_ref, c_ref, o_ref):
    c = c_ref[0, 0]
    lp = jnp.mean(jax.nn.softplus(-(xp_ref[...] + c)))
    ln = jnp.mean(jax.nn.softplus(xn_ref[...] + c))
    o_ref[...] = (lp + ln).reshape(1, 1)


def _run_d(xp2, xn2, c):
    return pl.pallas_call(
        _d_body,
        out_shape=jax.ShapeDtypeStruct((1, 1), jnp.float32),
    )(xp2, xn2, c)


def kernel(n_features, e_features, edge_index, W_apply_w, W_apply_b,
           W_edge_w, W_edge_b, disc_W):
    nf = jnp.concatenate(
        [n_features.reshape(_N, _DIN),
         jnp.zeros((_NP - _N, _DIN), jnp.float32)], axis=0)
    ef = e_features.reshape(_E, _EDIM)
    src = edge_index[0]
    dst = edge_index[1]
    perm = jnp.asarray(_PERM)

    spos, sneg, din, dout = _run_sc_scatter(ef, src, dst, perm)

    wnT = W_apply_w[:, :_DIN].T
    weT = W_apply_w[:, _DIN:].T
    brow = W_apply_b.reshape(1, _H)
    beb_row = W_edge_b.reshape(1, _EOUT)
    tab8, c = _run_b(nf, spos, sneg, din, dout,
                     wnT, weT, brow, W_edge_w, beb_row, disc_W)

    xp, xn = _run_sc_edge(src, dst, tab8[8:12].reshape(4 * _NP))

    loss = _run_d(xp.reshape(_E // _DIN, _DIN), xn.reshape(_E // _DIN, _DIN),
                  c)
    return loss[0, 0]


# speedup vs baseline: 1.0570x; 1.0169x over previous
import numpy as np
import jax
import jax.numpy as jnp
from jax import lax
from jax.experimental import pallas as pl
from jax.experimental.pallas import tpu as pltpu
from jax.experimental.pallas import tpu_sc as plsc

_N = 10000
_E = 320000
_DIN = 128
_EDIM = 16
_H = 128
_EOUT = 256

_NC = 2
_NS = 16
_NW = _NC * _NS
_NP = 10240
_ROWS_PT = _NP // _NS
_EPW = _E // _NW
_BLK = 400
_NBLK = _EPW // _BLK
_TN = 640


def _make_perm():
    try:
        try:
            dev = jax.local_devices(backend="cpu")[0]
            with jax.default_device(dev):
                p = np.asarray(jax.random.permutation(jax.random.key(1), _E))
        except Exception:
            p = np.asarray(jax.random.permutation(jax.random.key(1), _E))
    except Exception:
        p = np.arange(_E)
    return p.astype(np.int32)


_PERM = _make_perm()


def _sc_mesh():
    return plsc.VectorSubcoreMesh(core_axis_name="c", subcore_axis_name="s",
                                  num_cores=_NC, num_subcores=_NS)


def _sc_scatter_body(efeat, src, dst, perm, zeros_h, ones_h,
                     spos_o, sneg_o, din_o, dout_o,
                     feat0, feat1, featp0, featp1, ones_v, zeros_v,
                     src0, src1, dst0, dst1, perm0, perm1,
                     acc_pos, acc_neg, acc_din, acc_dout,
                     seml0, seml1, semc0, semc1, sem_g):
    cid = lax.axis_index("c")
    sid = lax.axis_index("s")
    wid = cid * _NS + sid

    pltpu.sync_copy(zeros_h, zeros_v)
    pltpu.sync_copy(ones_h, ones_v)

    row0 = sid * _ROWS_PT
    pltpu.sync_copy(zeros_v, acc_pos.at[pl.ds(row0, _ROWS_PT)])
    pltpu.sync_copy(zeros_v, acc_neg.at[pl.ds(row0, _ROWS_PT)])
    pltpu.sync_copy(zeros_v, acc_din.at[pl.ds(row0, _ROWS_PT)])
    pltpu.sync_copy(zeros_v, acc_dout.at[pl.ds(row0, _ROWS_PT)])
    plsc.subcore_barrier()

    slots = ((feat0, featp0, src0, dst0, perm0, seml0, semc0),
             (feat1, featp1, src1, dst1, perm1, seml1, semc1))

    def loads(b, s):
        fv, _, sv, dv, pv, seml, _ = slots[s]
        base = wid * _EPW + b * _BLK
        pltpu.async_copy(efeat.at[pl.ds(base, _BLK)], fv, seml)
        pltpu.async_copy(src.at[pl.ds(base, _BLK)], sv, seml)
        pltpu.async_copy(dst.at[pl.ds(base, _BLK)], dv, seml)
        pltpu.async_copy(perm.at[pl.ds(base, _BLK)], pv, seml)

    def wait_loads(b, s):
        fv, _, sv, dv, pv, seml, _ = slots[s]
        base = wid * _EPW + b * _BLK
        pltpu.make_async_copy(efeat.at[pl.ds(base, _BLK)], fv, seml).wait()
        pltpu.make_async_copy(src.at[pl.ds(base, _BLK)], sv, seml).wait()
        pltpu.make_async_copy(dst.at[pl.ds(base, _BLK)], dv, seml).wait()
        pltpu.make_async_copy(perm.at[pl.ds(base, _BLK)], pv, seml).wait()

    def wait_scat(s):
        fv, fpv, sv, dv, _, _, semc = slots[s]
        pltpu.make_async_copy(fv, acc_pos.at[dv], semc).wait()
        pltpu.make_async_copy(ones_v, acc_din.at[dv], semc).wait()
        pltpu.make_async_copy(ones_v, acc_dout.at[sv], semc).wait()
        pltpu.make_async_copy(fpv, acc_neg.at[dv], semc).wait()

    def half(b, s, t):
        fv, fpv, sv, dv, pv, _, semc = slots[s]
        wait_loads(b, s)
        g = pltpu.async_copy(efeat.at[pv], fpv, sem_g)
        pltpu.async_copy(fv, acc_pos.at[dv], semc, add=True)
        pltpu.async_copy(ones_v, acc_din.at[dv], semc, add=True)
        pltpu.async_copy(ones_v, acc_dout.at[sv], semc, add=True)

        @pl.when(b >= 1)
        def _w():
            wait_scat(t)

        @pl.when(b + 1 < _NBLK)
        def _l():
            loads(b + 1, t)

        g.wait()
        pltpu.async_copy(fpv, acc_neg.at[dv], semc, add=True)

    loads(0, 0)

    def g_body(g, carry):
        b0 = g * 2
        half(b0, 0, 1)

        @pl.when(b0 + 1 < _NBLK)
        def _h2():
            half(b0 + 1, 1, 0)

        return carry

    lax.fori_loop(0, (_NBLK + 1) // 2, g_body, 0)
    wait_scat((_NBLK - 1) % 2)
    plsc.subcore_barrier()

    rows = pl.ds(row0, _ROWS_PT)
    pltpu.sync_copy(acc_pos.at[rows], spos_o.at[cid, rows])
    pltpu.sync_copy(acc_neg.at[rows], sneg_o.at[cid, rows])
    pltpu.sync_copy(acc_din.at[rows], din_o.at[cid, rows])
    pltpu.sync_copy(acc_dout.at[rows], dout_o.at[cid, rows])


def _run_sc_scatter(ef, src, dst, perm):
    acc = jax.ShapeDtypeStruct((_NC, _NP, _EDIM), jnp.float32)
    f = pl.kernel(
        _sc_scatter_body,
        out_type=(acc, acc, acc, acc),
        mesh=_sc_mesh(),
        compiler_params=pltpu.CompilerParams(use_tc_tiling_on_sc=False),
        scratch_types=[
            pltpu.VMEM((_BLK, _EDIM), jnp.float32),
            pltpu.VMEM((_BLK, _EDIM), jnp.float32),
            pltpu.VMEM((_BLK, _EDIM), jnp.float32),
            pltpu.VMEM((_BLK, _EDIM), jnp.float32),
            pltpu.VMEM((_BLK, _EDIM), jnp.float32),
            pltpu.VMEM((_ROWS_PT, _EDIM), jnp.float32),
            pltpu.VMEM((_BLK,), jnp.int32),
            pltpu.VMEM((_BLK,), jnp.int32),
            pltpu.VMEM((_BLK,), jnp.int32),
            pltpu.VMEM((_BLK,), jnp.int32),
            pltpu.VMEM((_BLK,), jnp.int32),
            pltpu.VMEM((_BLK,), jnp.int32),
            pltpu.VMEM_SHARED((_NP, _EDIM), jnp.float32),
            pltpu.VMEM_SHARED((_NP, _EDIM), jnp.float32),
            pltpu.VMEM_SHARED((_NP, _EDIM), jnp.float32),
            pltpu.VMEM_SHARED((_NP, _EDIM), jnp.float32),
            pltpu.SemaphoreType.DMA,
            pltpu.SemaphoreType.DMA,
            pltpu.SemaphoreType.DMA,
            pltpu.SemaphoreType.DMA,
            pltpu.SemaphoreType.DMA,
        ],
    )
    zeros_h = jnp.zeros((_ROWS_PT, _EDIM), jnp.float32)
    ones_h = jnp.ones((_BLK, _EDIM), jnp.float32)
    return f(ef, src, dst, perm, zeros_h, ones_h)


def _b_body(nf_ref, spos_ref, sneg_ref, din_ref, dout_ref,
            wnT_ref, weT_ref, b_ref, wew_ref, beb_ref, dw_ref,
            tab_ref, c_ref, hp_s, hn_s, acc_ref, u_s):
    p = pl.program_id(0)
    i = pl.program_id(1)
    off = pl.multiple_of(i * _TN, _TN)

    @pl.when(p == 0)
    def _phase0():
        din = din_ref[0, :, 0:1] + din_ref[1, :, 0:1]
        dout = dout_ref[0, :, 0:1] + dout_ref[1, :, 0:1]
        inv = 1.0 / jnp.maximum(din, 1.0)
        sp = (spos_ref[0] + spos_ref[1]) * inv
        sn = (sneg_ref[0] + sneg_ref[1]) * inv
        base = jnp.dot(nf_ref[...], wnT_ref[...],
                       preferred_element_type=jnp.float32) + b_ref[...]
        weT = weT_ref[...]
        hp = jnp.maximum(
            base + jnp.dot(sp, weT, preferred_element_type=jnp.float32), 0.0)
        hn = jnp.maximum(
            base + jnp.dot(sn, weT, preferred_element_type=jnp.float32), 0.0)
        hp_s[pl.ds(off, _TN), :] = hp
        hn_s[pl.ds(off, _TN), :] = hn

        mask = ((lax.broadcasted_iota(jnp.int32, (_TN, 1), 0) + i * _TN)
                < _N).astype(jnp.float32)
        msrc_t = jnp.sum(hp * (dout * mask), axis=0, keepdims=True)
        mdst_t = jnp.sum(hp * (din * mask), axis=0, keepdims=True)

        @pl.when(i == 0)
        def _init():
            acc_ref[...] = jnp.zeros((2, _H), jnp.float32)

        acc_ref[0:1, :] += msrc_t
        acc_ref[1:2, :] += mdst_t

        @pl.when(i == (_NP // _TN) - 1)
        def _fin():
            msrc = acc_ref[0:1, :]
            mdst = acc_ref[1:2, :]
            wew = wew_ref[...]
            w1 = wew[:, :_H]
            w2 = wew[:, _H:]
            dims = (((1,), (1,)), ((), ()))
            me = (lax.dot_general(msrc, w1, dims,
                                  preferred_element_type=jnp.float32)
                  + lax.dot_general(mdst, w2, dims,
                                    preferred_element_type=jnp.float32)
                  ) * (1.0 / _E) + beb_ref[...]
            summ = jax.nn.sigmoid(me)
            ws = lax.dot_general(summ, dw_ref[...], dims,
                                 preferred_element_type=jnp.float32)
            dims2 = (((1,), (0,)), ((), ()))
            u1 = lax.dot_general(ws, w1, dims2,
                                 preferred_element_type=jnp.float32)
            u2 = lax.dot_general(ws, w2, dims2,
                                 preferred_element_type=jnp.float32)
            u_s[...] = jnp.concatenate([u1, u2], axis=0)
            c_ref[...] = jnp.sum(beb_ref[...] * ws).reshape(1, 1)

    @pl.when(p == 1)
    def _phase1():
        u = u_s[...]
        hp = hp_s[pl.ds(off, _TN), :]
        hn = hn_s[pl.ds(off, _TN), :]
        dims = (((1,), (1,)), ((), ()))
        pq_p = lax.dot_general(u, hp, dims,
                               preferred_element_type=jnp.float32)
        pq_n = lax.dot_general(u, hn, dims,
                               preferred_element_type=jnp.float32)
        tab_ref[...] = jnp.concatenate(
            [pq_p, pq_n, jnp.zeros((4, _TN), jnp.float32)], axis=0)


def _run_b(nf, spos, sneg, din, dout, wnT, weT, brow, wew, beb_row, dw):
    grid = (2, _NP // _TN)

    def _i0(p, i):
        return jnp.where(p == 0, i, 0)

    seg = pl.BlockSpec((2, _TN, _EDIM), lambda p, i: (0, _i0(p, i), 0))
    return pl.pallas_call(
        _b_body,
        grid=grid,
        in_specs=[
            pl.BlockSpec((_TN, _DIN), lambda p, i: (_i0(p, i), 0)),
            seg, seg, seg, seg,
            pl.BlockSpec((_DIN, _H), lambda p, i: (0, 0)),
            pl.BlockSpec((_EDIM, _H), lambda p, i: (0, 0)),
            pl.BlockSpec((1, _H), lambda p, i: (0, 0)),
            pl.BlockSpec((_EOUT, _EOUT), lambda p, i: (0, 0)),
            pl.BlockSpec((1, _EOUT), lambda p, i: (0, 0)),
            pl.BlockSpec((_EOUT, _EOUT), lambda p, i: (0, 0)),
        ],
        out_specs=[pl.BlockSpec((8, _TN), lambda p, i: (p, i)),
                   pl.BlockSpec((1, 1), lambda p, i: (0, 0))],
        out_shape=[jax.ShapeDtypeStruct((16, _NP), jnp.float32),
                   jax.ShapeDtypeStruct((1, 1), jnp.float32)],
        scratch_shapes=[pltpu.VMEM((_NP, _H), jnp.float32),
                        pltpu.VMEM((_NP, _H), jnp.float32),
                        pltpu.VMEM((2, _H), jnp.float32),
                        pltpu.VMEM((2, _H), jnp.float32)],
        compiler_params=pltpu.CompilerParams(
            vmem_limit_bytes=100 * 1024 * 1024),
    )(nf, spos, sneg, din, dout, wnT, weT, brow, wew, beb_row, dw)


def _sc_edge_body(src, dst, tab, xp_o, xn_o,
                  tab_v, src_v, dst_v, xp_v, xn_v):
    cid = lax.axis_index("c")
    sid = lax.axis_index("s")
    wid = cid * _NS + sid

    pltpu.sync_copy(tab, tab_v)
    base = wid * _EPW
    pltpu.sync_copy(src.at[pl.ds(base, _EPW)], src_v)
    pltpu.sync_copy(dst.at[pl.ds(base, _EPW)], dst_v)

    def it(i, carry):
        s = pl.ds(i * 16, 16)
        sv = src_v[s]
        dv = dst_v[s]
        xp_v[s] = (plsc.load_gather(tab_v, [sv])
                   + plsc.load_gather(tab_v, [dv + _NP]))
        xn_v[s] = (plsc.load_gather(tab_v, [sv + 2 * _NP])
                   + plsc.load_gather(tab_v, [dv + 3 * _NP]))
        return carry

    lax.fori_loop(0, _EPW // 16, it, 0)
    pltpu.sync_copy(xp_v, xp_o.at[pl.ds(base, _EPW)])
    pltpu.sync_copy(xn_v, xn_o.at[pl.ds(base, _EPW)])


def _run_sc_edge(src, dst, tab):
    out = jax.ShapeDtypeStruct((_E,), jnp.float32)
    f = pl.kernel(
        _sc_edge_body,
        out_type=(out, out),
        mesh=_sc_mesh(),
        compiler_params=pltpu.CompilerParams(use_tc_tiling_on_sc=False,
                                             needs_layout_passes=False),
        scratch_types=[
            pltpu.VMEM((4 * _NP,), jnp.float32),
            pltpu.VMEM((_EPW,), jnp.int32),
            pltpu.VMEM((_EPW,), jnp.int32),
            pltpu.VMEM((_EPW,), jnp.float32),
            pltpu.VMEM((_EPW,), jnp.float32),
        ],
    )
    return f(src, dst, tab)


def _d_body(xp_ref, xn_ref, c_ref, o_ref):
    c = c_ref[0, 0]
    lp = jnp.mean(jax.nn.softplus(-(xp_ref[...] + c)))
    ln = jnp.mean(jax.nn.softplus(xn_ref[...] + c))
    o_ref[...] = (lp + ln).reshape(1, 1)


def _run_d(xp2, xn2, c):
    return pl.pallas_call(
        _d_body,
        out_shape=jax.ShapeDtypeStruct((1, 1), jnp.float32),
    )(xp2, xn2, c)


def kernel(n_features, e_features, edge_index, W_apply_w, W_apply_b,
           W_edge_w, W_edge_b, disc_W):
    nf = n_features.reshape(_N, _DIN)
    ef = e_features.reshape(_E, _EDIM)
    src = edge_index[0]
    dst = edge_index[1]
    perm = jnp.asarray(_PERM)

    spos, sneg, din, dout = _run_sc_scatter(ef, src, dst, perm)

    wnT = W_apply_w[:, :_DIN].T
    weT = W_apply_w[:, _DIN:].T
    brow = W_apply_b.reshape(1, _H)
    beb_row = W_edge_b.reshape(1, _EOUT)
    tab8, c = _run_b(nf, spos, sneg, din, dout,
                     wnT, weT, brow, W_edge_w, beb_row, disc_W)

    xp, xn = _run_sc_edge(src, dst, tab8[8:12].reshape(4 * _NP))

    loss = _run_d(xp.reshape(_E // _DIN, _DIN), xn.reshape(_E // _DIN, _DIN),
                  c)
    return loss[0, 0]


# speedup vs baseline: 1.3420x; 1.2696x over previous
import numpy as np
import jax
import jax.numpy as jnp
from jax import lax
from jax.experimental import pallas as pl
from jax.experimental.pallas import tpu as pltpu
from jax.experimental.pallas import tpu_sc as plsc

_N = 10000
_E = 320000
_DIN = 128
_EDIM = 16
_H = 128
_EOUT = 256

_NC = 2
_NS = 16
_NW = _NC * _NS
_NP = 10240
_ROWS_PT = _NP // _NS
_EPW = _E // _NW
_BLK = 400
_NBLK = _EPW // _BLK
_TN = 640


def _make_perm():
    try:
        try:
            dev = jax.local_devices(backend="cpu")[0]
            with jax.default_device(dev):
                p = np.asarray(jax.random.permutation(jax.random.key(1), _E))
        except Exception:
            p = np.asarray(jax.random.permutation(jax.random.key(1), _E))
    except Exception:
        p = np.arange(_E)
    return p.astype(np.int32)


_PERM = _make_perm()
_INV = np.empty_like(_PERM)
_INV[_PERM] = np.arange(_E, dtype=np.int32)


def _sc_mesh():
    return plsc.VectorSubcoreMesh(core_axis_name="c", subcore_axis_name="s",
                                  num_cores=_NC, num_subcores=_NS)


def _sc_scatter_body(efeat_fm, src, dst, dstn, zeros_h, ones_h,
                     spos_o, sneg_o, din_o, dout_o,
                     col0, col1, row0v, row1v, ones_v, zeros_v,
                     src0, src1, dst0, dst1, dn0, dn1,
                     acc_pos, acc_neg, acc_din, acc_dout,
                     seml0, seml1, semc0, semc1):
    cid = lax.axis_index("c")
    sid = lax.axis_index("s")
    wid = cid * _NS + sid

    pltpu.sync_copy(zeros_h, zeros_v)
    pltpu.sync_copy(ones_h, ones_v)

    rr = sid * _ROWS_PT
    pltpu.sync_copy(zeros_v, acc_pos.at[pl.ds(rr, _ROWS_PT)])
    pltpu.sync_copy(zeros_v, acc_neg.at[pl.ds(rr, _ROWS_PT)])
    pltpu.sync_copy(zeros_v, acc_din.at[pl.ds(rr, _ROWS_PT)])
    pltpu.sync_copy(zeros_v, acc_dout.at[pl.ds(rr, _ROWS_PT)])
    plsc.subcore_barrier()

    slots = ((col0, row0v, src0, dst0, dn0, seml0, semc0),
             (col1, row1v, src1, dst1, dn1, seml1, semc1))
    lane = jax.lax.iota(jnp.int32, 16)

    def loads(b, s):
        cv, _, sv, dv, nv, seml, _ = slots[s]
        base = wid * _EPW + b * _BLK
        pltpu.async_copy(efeat_fm.at[:, pl.ds(base, _BLK)], cv, seml)
        pltpu.async_copy(src.at[pl.ds(base, _BLK)], sv, seml)
        pltpu.async_copy(dst.at[pl.ds(base, _BLK)], dv, seml)
        pltpu.async_copy(dstn.at[pl.ds(base, _BLK)], nv, seml)

    def wait_loads(b, s):
        cv, _, sv, dv, nv, seml, _ = slots[s]
        base = wid * _EPW + b * _BLK
        pltpu.make_async_copy(efeat_fm.at[:, pl.ds(base, _BLK)], cv,
                              seml).wait()
        pltpu.make_async_copy(src.at[pl.ds(base, _BLK)], sv, seml).wait()
        pltpu.make_async_copy(dst.at[pl.ds(base, _BLK)], dv, seml).wait()
        pltpu.make_async_copy(dstn.at[pl.ds(base, _BLK)], nv, seml).wait()

    def transpose(s):
        cv, rv, _, _, _, _, _ = slots[s]

        def tbody(j, carry):
            rows = lane + j * 16
            for k in range(_EDIM):
                v = cv[k, pl.ds(j * 16, 16)]
                plsc.store_scatter(
                    rv, [rows, jnp.full((16,), k, jnp.int32)], v)
            return carry

        lax.fori_loop(0, _BLK // 16, tbody, 0)

    def wait_scat(s):
        _, rv, sv, dv, nv, _, semc = slots[s]
        pltpu.make_async_copy(rv, acc_pos.at[dv], semc).wait()
        pltpu.make_async_copy(rv, acc_neg.at[nv], semc).wait()
        pltpu.make_async_copy(ones_v, acc_din.at[dv], semc).wait()
        pltpu.make_async_copy(ones_v, acc_dout.at[sv], semc).wait()

    def half(b, s, t):
        _, rv, sv, dv, nv, _, semc = slots[s]
        wait_loads(b, s)
        transpose(s)
        pltpu.async_copy(rv, acc_pos.at[dv], semc, add=True)
        pltpu.async_copy(rv, acc_neg.at[nv], semc, add=True)
        pltpu.async_copy(ones_v, acc_din.at[dv], semc, add=True)
        pltpu.async_copy(ones_v, acc_dout.at[sv], semc, add=True)

        @pl.when(b >= 1)
        def _w():
            wait_scat(t)

        @pl.when(b + 1 < _NBLK)
        def _l():
            loads(b + 1, t)

    loads(0, 0)

    def g_body(g, carry):
        b0 = g * 2
        half(b0, 0, 1)

        @pl.when(b0 + 1 < _NBLK)
        def _h2():
            half(b0 + 1, 1, 0)

        return carry

    lax.fori_loop(0, (_NBLK + 1) // 2, g_body, 0)
    wait_scat((_NBLK - 1) % 2)
    plsc.subcore_barrier()

    rows = pl.ds(rr, _ROWS_PT)
    pltpu.sync_copy(acc_pos.at[rows], spos_o.at[cid, rows])
    pltpu.sync_copy(acc_neg.at[rows], sneg_o.at[cid, rows])
    pltpu.sync_copy(acc_din.at[rows], din_o.at[cid, rows])
    pltpu.sync_copy(acc_dout.at[rows], dout_o.at[cid, rows])


def _run_sc_scatter(ef_fm, src, dst, dstn):
    acc = jax.ShapeDtypeStruct((_NC, _NP, _EDIM), jnp.float32)
    f = pl.kernel(
        _sc_scatter_body,
        out_type=(acc, acc, acc, acc),
        mesh=_sc_mesh(),
        compiler_params=pltpu.CompilerParams(use_tc_tiling_on_sc=False,
                                             needs_layout_passes=False),
        scratch_types=[
            pltpu.VMEM((_EDIM, _BLK), jnp.float32),
            pltpu.VMEM((_EDIM, _BLK), jnp.float32),
            pltpu.VMEM((_BLK, _EDIM), jnp.float32),
            pltpu.VMEM((_BLK, _EDIM), jnp.float32),
            pltpu.VMEM((_BLK, _EDIM), jnp.float32),
            pltpu.VMEM((_ROWS_PT, _EDIM), jnp.float32),
            pltpu.VMEM((_BLK,), jnp.int32),
            pltpu.VMEM((_BLK,), jnp.int32),
            pltpu.VMEM((_BLK,), jnp.int32),
            pltpu.VMEM((_BLK,), jnp.int32),
            pltpu.VMEM((_BLK,), jnp.int32),
            pltpu.VMEM((_BLK,), jnp.int32),
            pltpu.VMEM_SHARED((_NP, _EDIM), jnp.float32),
            pltpu.VMEM_SHARED((_NP, _EDIM), jnp.float32),
            pltpu.VMEM_SHARED((_NP, _EDIM), jnp.float32),
            pltpu.VMEM_SHARED((_NP, _EDIM), jnp.float32),
            pltpu.SemaphoreType.DMA,
            pltpu.SemaphoreType.DMA,
            pltpu.SemaphoreType.DMA,
            pltpu.SemaphoreType.DMA,
        ],
    )
    zeros_h = jnp.zeros((_ROWS_PT, _EDIM), jnp.float32)
    ones_h = jnp.ones((_BLK, _EDIM), jnp.float32)
    return f(ef_fm, src, dst, dstn, zeros_h, ones_h)


def _b_body(nf_ref, spos_ref, sneg_ref, din_ref, dout_ref,
            wnT_ref, weT_ref, b_ref, wew_ref, beb_ref, dw_ref,
            tab_ref, c_ref, hp_s, hn_s, acc_ref, u_s):
    p = pl.program_id(0)
    i = pl.program_id(1)
    off = pl.multiple_of(i * _TN, _TN)

    @pl.when(p == 0)
    def _phase0():
        din = din_ref[0, :, 0:1] + din_ref[1, :, 0:1]
        dout = dout_ref[0, :, 0:1] + dout_ref[1, :, 0:1]
        inv = 1.0 / jnp.maximum(din, 1.0)
        sp = (spos_ref[0] + spos_ref[1]) * inv
        sn = (sneg_ref[0] + sneg_ref[1]) * inv
        base = jnp.dot(nf_ref[...], wnT_ref[...],
                       preferred_element_type=jnp.float32) + b_ref[...]
        weT = weT_ref[...]
        hp = jnp.maximum(
            base + jnp.dot(sp, weT, preferred_element_type=jnp.float32), 0.0)
        hn = jnp.maximum(
            base + jnp.dot(sn, weT, preferred_element_type=jnp.float32), 0.0)
        hp_s[pl.ds(off, _TN), :] = hp
        hn_s[pl.ds(off, _TN), :] = hn

        mask = ((lax.broadcasted_iota(jnp.int32, (_TN, 1), 0) + i * _TN)
                < _N).astype(jnp.float32)
        msrc_t = jnp.sum(hp * (dout * mask), axis=0, keepdims=True)
        mdst_t = jnp.sum(hp * (din * mask), axis=0, keepdims=True)

        @pl.when(i == 0)
        def _init():
            acc_ref[...] = jnp.zeros((2, _H), jnp.float32)

        acc_ref[0:1, :] += msrc_t
        acc_ref[1:2, :] += mdst_t

        @pl.when(i == (_NP // _TN) - 1)
        def _fin():
            msrc = acc_ref[0:1, :]
            mdst = acc_ref[1:2, :]
            wew = wew_ref[...]
            w1 = wew[:, :_H]
            w2 = wew[:, _H:]
            dims = (((1,), (1,)), ((), ()))
            me = (lax.dot_general(msrc, w1, dims,
                                  preferred_element_type=jnp.float32)
                  + lax.dot_general(mdst, w2, dims,
                                    preferred_element_type=jnp.float32)
                  ) * (1.0 / _E) + beb_ref[...]
            summ = jax.nn.sigmoid(me)
            ws = lax.dot_general(summ, dw_ref[...], dims,
                                 preferred_element_type=jnp.float32)
            dims2 = (((1,), (0,)), ((), ()))
            u1 = lax.dot_general(ws, w1, dims2,
                                 preferred_element_type=jnp.float32)
            u2 = lax.dot_general(ws, w2, dims2,
                                 preferred_element_type=jnp.float32)
            u_s[...] = jnp.concatenate([u1, u2], axis=0)
            c_ref[...] = jnp.sum(beb_ref[...] * ws).reshape(1, 1)

    @pl.when(p == 1)
    def _phase1():
        u = u_s[...]
        hp = hp_s[pl.ds(off, _TN), :]
        hn = hn_s[pl.ds(off, _TN), :]
        dims = (((1,), (1,)), ((), ()))
        pq_p = lax.dot_general(u, hp, dims,
                               preferred_element_type=jnp.float32)
        pq_n = lax.dot_general(u, hn, dims,
                               preferred_element_type=jnp.float32)
        tab_ref[...] = jnp.concatenate(
            [pq_p, pq_n, jnp.zeros((4, _TN), jnp.float32)], axis=0)


def _run_b(nf, spos, sneg, din, dout, wnT, weT, brow, wew, beb_row, dw):
    grid = (2, _NP // _TN)

    def _i0(p, i):
        return jnp.where(p == 0, i, 0)

    seg = pl.BlockSpec((2, _TN, _EDIM), lambda p, i: (0, _i0(p, i), 0))
    return pl.pallas_call(
        _b_body,
        grid=grid,
        in_specs=[
            pl.BlockSpec((_TN, _DIN), lambda p, i: (_i0(p, i), 0)),
            seg, seg, seg, seg,
            pl.BlockSpec((_DIN, _H), lambda p, i: (0, 0)),
            pl.BlockSpec((_EDIM, _H), lambda p, i: (0, 0)),
            pl.BlockSpec((1, _H), lambda p, i: (0, 0)),
            pl.BlockSpec((_EOUT, _EOUT), lambda p, i: (0, 0)),
            pl.BlockSpec((1, _EOUT), lambda p, i: (0, 0)),
            pl.BlockSpec((_EOUT, _EOUT), lambda p, i: (0, 0)),
        ],
        out_specs=[pl.BlockSpec((8, _TN), lambda p, i: (p, i)),
                   pl.BlockSpec((1, 1), lambda p, i: (0, 0))],
        out_shape=[jax.ShapeDtypeStruct((16, _NP), jnp.float32),
                   jax.ShapeDtypeStruct((1, 1), jnp.float32)],
        scratch_shapes=[pltpu.VMEM((_NP, _H), jnp.float32),
                        pltpu.VMEM((_NP, _H), jnp.float32),
                        pltpu.VMEM((2, _H), jnp.float32),
                        pltpu.VMEM((2, _H), jnp.float32)],
        compiler_params=pltpu.CompilerParams(
            vmem_limit_bytes=100 * 1024 * 1024),
    )(nf, spos, sneg, din, dout, wnT, weT, brow, wew, beb_row, dw)


def _sc_edge_body(src, dst, tab, xp_o, xn_o,
                  tab_v, src_v, dst_v, xp_v, xn_v):
    cid = lax.axis_index("c")
    sid = lax.axis_index("s")
    wid = cid * _NS + sid

    pltpu.sync_copy(tab, tab_v)
    base = wid * _EPW
    pltpu.sync_copy(src.at[pl.ds(base, _EPW)], src_v)
    pltpu.sync_copy(dst.at[pl.ds(base, _EPW)], dst_v)

    def it(i, carry):
        s = pl.ds(i * 16, 16)
        sv = src_v[s]
        dv = dst_v[s]
        xp_v[s] = (plsc.load_gather(tab_v, [sv])
                   + plsc.load_gather(tab_v, [dv + _NP]))
        xn_v[s] = (plsc.load_gather(tab_v, [sv + 2 * _NP])
                   + plsc.load_gather(tab_v, [dv + 3 * _NP]))
        return carry

    lax.fori_loop(0, _EPW // 16, it, 0)
    pltpu.sync_copy(xp_v, xp_o.at[pl.ds(base, _EPW)])
    pltpu.sync_copy(xn_v, xn_o.at[pl.ds(base, _EPW)])


def _run_sc_edge(src, dst, tab):
    out = jax.ShapeDtypeStruct((_E,), jnp.float32)
    f = pl.kernel(
        _sc_edge_body,
        out_type=(out, out),
        mesh=_sc_mesh(),
        compiler_params=pltpu.CompilerParams(use_tc_tiling_on_sc=False,
                                             needs_layout_passes=False),
        scratch_types=[
            pltpu.VMEM((4 * _NP,), jnp.float32),
            pltpu.VMEM((_EPW,), jnp.int32),
            pltpu.VMEM((_EPW,), jnp.int32),
            pltpu.VMEM((_EPW,), jnp.float32),
            pltpu.VMEM((_EPW,), jnp.float32),
        ],
    )
    return f(src, dst, tab)


def _d_body(xp_ref, xn_ref, c_ref, o_ref):
    c = c_ref[0, 0]
    lp = jnp.mean(jax.nn.softplus(-(xp_ref[...] + c)))
    ln = jnp.mean(jax.nn.softplus(xn_ref[...] + c))
    o_ref[...] = (lp + ln).reshape(1, 1)


def _run_d(xp2, xn2, c):
    return pl.pallas_call(
        _d_body,
        out_shape=jax.ShapeDtypeStruct((1, 1), jnp.float32),
    )(xp2, xn2, c)


def kernel(n_features, e_features, edge_index, W_apply_w, W_apply_b,
           W_edge_w, W_edge_b, disc_W):
    nf = n_features.reshape(_N, _DIN)
    ef_fm = e_features.reshape(_E, _EDIM).T
    src = edge_index[0]
    dst = edge_index[1]
    dstn = jnp.take(dst, jnp.asarray(_INV))

    spos, sneg, din, dout = _run_sc_scatter(ef_fm, src, dst, dstn)

    wnT = W_apply_w[:, :_DIN].T
    weT = W_apply_w[:, _DIN:].T
    brow = W_apply_b.reshape(1, _H)
    beb_row = W_edge_b.reshape(1, _EOUT)
    tab8, c = _run_b(nf, spos, sneg, din, dout,
                     wnT, weT, brow, W_edge_w, beb_row, disc_W)

    xp, xn = _run_sc_edge(src, dst, tab8[8:12].reshape(4 * _NP))

    loss = _run_d(xp.reshape(_E // _DIN, _DIN), xn.reshape(_E // _DIN, _DIN),
                  c)
    return loss[0, 0]


# speedup vs baseline: 1.3510x; 1.0067x over previous
import numpy as np
import jax
import jax.numpy as jnp
from jax import lax
from jax.experimental import pallas as pl
from jax.experimental.pallas import tpu as pltpu
from jax.experimental.pallas import tpu_sc as plsc

_N = 10000
_E = 320000
_DIN = 128
_EDIM = 16
_H = 128
_EOUT = 256

_NC = 2
_NS = 16
_NW = _NC * _NS
_NP = 10240
_ROWS_PT = _NP // _NS
_EPW = _E // _NW
_BLK = 400
_NBLK = _EPW // _BLK
_TN = 640


def _make_perm():
    try:
        try:
            dev = jax.local_devices(backend="cpu")[0]
            with jax.default_device(dev):
                p = np.asarray(jax.random.permutation(jax.random.key(1), _E))
        except Exception:
            p = np.asarray(jax.random.permutation(jax.random.key(1), _E))
    except Exception:
        p = np.arange(_E)
    return p.astype(np.int32)


_PERM = _make_perm()
_INV = np.empty_like(_PERM)
_INV[_PERM] = np.arange(_E, dtype=np.int32)


def _sc_mesh():
    return plsc.VectorSubcoreMesh(core_axis_name="c", subcore_axis_name="s",
                                  num_cores=_NC, num_subcores=_NS)


def _sc_scatter_body(efeat_fm, src, dst, dstn, zeros_h, ones_h,
                     spos_o, sneg_o, din_o, dout_o,
                     col0, col1, row0v, row1v, ones_v, zeros_v,
                     src0, src1, dst0, dst1, dn0, dn1,
                     acc_pos, acc_neg, acc_din, acc_dout,
                     seml0, seml1, semc0, semc1):
    cid = lax.axis_index("c")
    sid = lax.axis_index("s")
    wid = cid * _NS + sid

    pltpu.sync_copy(zeros_h, zeros_v)
    pltpu.sync_copy(ones_h, ones_v)

    rr = sid * _ROWS_PT
    pltpu.sync_copy(zeros_v, acc_pos.at[pl.ds(rr, _ROWS_PT)])
    pltpu.sync_copy(zeros_v, acc_neg.at[pl.ds(rr, _ROWS_PT)])
    pltpu.sync_copy(zeros_v, acc_din.at[pl.ds(rr, _ROWS_PT)])
    pltpu.sync_copy(zeros_v, acc_dout.at[pl.ds(rr, _ROWS_PT)])
    plsc.subcore_barrier()

    slots = ((col0, row0v, src0, dst0, dn0, seml0, semc0),
             (col1, row1v, src1, dst1, dn1, seml1, semc1))
    lane = jax.lax.iota(jnp.int32, 16)

    def loads(b, s):
        cv, _, sv, dv, nv, seml, _ = slots[s]
        base = wid * _EPW + b * _BLK
        pltpu.async_copy(efeat_fm.at[:, pl.ds(base, _BLK)], cv, seml)
        pltpu.async_copy(src.at[pl.ds(base, _BLK)], sv, seml)
        pltpu.async_copy(dst.at[pl.ds(base, _BLK)], dv, seml)
        pltpu.async_copy(dstn.at[pl.ds(base, _BLK)], nv, seml)

    def wait_loads(b, s):
        cv, _, sv, dv, nv, seml, _ = slots[s]
        base = wid * _EPW + b * _BLK
        pltpu.make_async_copy(efeat_fm.at[:, pl.ds(base, _BLK)], cv,
                              seml).wait()
        pltpu.make_async_copy(src.at[pl.ds(base, _BLK)], sv, seml).wait()
        pltpu.make_async_copy(dst.at[pl.ds(base, _BLK)], dv, seml).wait()
        pltpu.make_async_copy(dstn.at[pl.ds(base, _BLK)], nv, seml).wait()

    def transpose(s):
        cv, rv, _, _, _, _, _ = slots[s]

        def tbody(j, carry):
            rows = lane + j * 16
            for k in range(_EDIM):
                v = cv[k, pl.ds(j * 16, 16)]
                plsc.store_scatter(
                    rv, [rows, jnp.full((16,), k, jnp.int32)], v)
            return carry

        lax.fori_loop(0, _BLK // 16, tbody, 0)

    def wait_scat(s):
        _, rv, sv, dv, nv, _, semc = slots[s]
        pltpu.make_async_copy(rv, acc_pos.at[dv], semc).wait()
        pltpu.make_async_copy(rv, acc_neg.at[nv], semc).wait()
        pltpu.make_async_copy(ones_v, acc_din.at[dv], semc).wait()
        pltpu.make_async_copy(ones_v, acc_dout.at[sv], semc).wait()

    def half(b, s, t):
        _, rv, sv, dv, nv, _, semc = slots[s]
        wait_loads(b, s)
        transpose(s)
        pltpu.async_copy(rv, acc_pos.at[dv], semc, add=True)
        pltpu.async_copy(rv, acc_neg.at[nv], semc, add=True)
        pltpu.async_copy(ones_v, acc_din.at[dv], semc, add=True)
        pltpu.async_copy(ones_v, acc_dout.at[sv], semc, add=True)

        @pl.when(b >= 1)
        def _w():
            wait_scat(t)

        @pl.when(b + 1 < _NBLK)
        def _l():
            loads(b + 1, t)

    loads(0, 0)

    def g_body(g, carry):
        b0 = g * 2
        half(b0, 0, 1)

        @pl.when(b0 + 1 < _NBLK)
        def _h2():
            half(b0 + 1, 1, 0)

        return carry

    lax.fori_loop(0, (_NBLK + 1) // 2, g_body, 0)
    wait_scat((_NBLK - 1) % 2)
    plsc.subcore_barrier()

    rows = pl.ds(rr, _ROWS_PT)
    pltpu.sync_copy(acc_pos.at[rows], spos_o.at[cid, rows])
    pltpu.sync_copy(acc_neg.at[rows], sneg_o.at[cid, rows])
    pltpu.sync_copy(acc_din.at[rows], din_o.at[cid, rows])
    pltpu.sync_copy(acc_dout.at[rows], dout_o.at[cid, rows])


def _run_sc_scatter(ef_fm, src, dst, dstn):
    acc = jax.ShapeDtypeStruct((_NC, _NP, _EDIM), jnp.float32)
    f = pl.kernel(
        _sc_scatter_body,
        out_type=(acc, acc, acc, acc),
        mesh=_sc_mesh(),
        compiler_params=pltpu.CompilerParams(use_tc_tiling_on_sc=False,
                                             needs_layout_passes=False),
        scratch_types=[
            pltpu.VMEM((_EDIM, _BLK), jnp.float32),
            pltpu.VMEM((_EDIM, _BLK), jnp.float32),
            pltpu.VMEM((_BLK, _EDIM), jnp.float32),
            pltpu.VMEM((_BLK, _EDIM), jnp.float32),
            pltpu.VMEM((_BLK, _EDIM), jnp.float32),
            pltpu.VMEM((_ROWS_PT, _EDIM), jnp.float32),
            pltpu.VMEM((_BLK,), jnp.int32),
            pltpu.VMEM((_BLK,), jnp.int32),
            pltpu.VMEM((_BLK,), jnp.int32),
            pltpu.VMEM((_BLK,), jnp.int32),
            pltpu.VMEM((_BLK,), jnp.int32),
            pltpu.VMEM((_BLK,), jnp.int32),
            pltpu.VMEM_SHARED((_NP, _EDIM), jnp.float32),
            pltpu.VMEM_SHARED((_NP, _EDIM), jnp.float32),
            pltpu.VMEM_SHARED((_NP, _EDIM), jnp.float32),
            pltpu.VMEM_SHARED((_NP, _EDIM), jnp.float32),
            pltpu.SemaphoreType.DMA,
            pltpu.SemaphoreType.DMA,
            pltpu.SemaphoreType.DMA,
            pltpu.SemaphoreType.DMA,
        ],
    )
    zeros_h = jnp.zeros((_ROWS_PT, _EDIM), jnp.float32)
    ones_h = jnp.ones((_BLK, _EDIM), jnp.float32)
    return f(ef_fm, src, dst, dstn, zeros_h, ones_h)


def _b_body(nf_ref, spos_ref, sneg_ref, din_ref, dout_ref,
            wnT_ref, weT_ref, b_ref, wew_ref, beb_ref, dw_ref,
            tab_ref, c_ref, hp_s, hn_s, acc_ref, u_s):
    p = pl.program_id(0)
    i = pl.program_id(1)
    off = pl.multiple_of(i * _TN, _TN)

    @pl.when(p == 0)
    def _phase0():
        din = din_ref[0, :, 0:1] + din_ref[1, :, 0:1]
        dout = dout_ref[0, :, 0:1] + dout_ref[1, :, 0:1]
        inv = 1.0 / jnp.maximum(din, 1.0)
        sp = (spos_ref[0] + spos_ref[1]) * inv
        sn = (sneg_ref[0] + sneg_ref[1]) * inv
        base = jnp.dot(nf_ref[...], wnT_ref[...],
                       preferred_element_type=jnp.float32) + b_ref[...]
        weT = weT_ref[...]
        hp = jnp.maximum(
            base + jnp.dot(sp, weT, preferred_element_type=jnp.float32), 0.0)
        hn = jnp.maximum(
            base + jnp.dot(sn, weT, preferred_element_type=jnp.float32), 0.0)
        hp_s[pl.ds(off, _TN), :] = hp
        hn_s[pl.ds(off, _TN), :] = hn

        mask = ((lax.broadcasted_iota(jnp.int32, (_TN, 1), 0) + i * _TN)
                < _N).astype(jnp.float32)
        msrc_t = jnp.sum(hp * (dout * mask), axis=0, keepdims=True)
        mdst_t = jnp.sum(hp * (din * mask), axis=0, keepdims=True)

        @pl.when(i == 0)
        def _init():
            acc_ref[...] = jnp.zeros((2, _H), jnp.float32)

        acc_ref[0:1, :] += msrc_t
        acc_ref[1:2, :] += mdst_t

        @pl.when(i == (_NP // _TN) - 1)
        def _fin():
            msrc = acc_ref[0:1, :]
            mdst = acc_ref[1:2, :]
            wew = wew_ref[...]
            w1 = wew[:, :_H]
            w2 = wew[:, _H:]
            dims = (((1,), (1,)), ((), ()))
            me = (lax.dot_general(msrc, w1, dims,
                                  preferred_element_type=jnp.float32)
                  + lax.dot_general(mdst, w2, dims,
                                    preferred_element_type=jnp.float32)
                  ) * (1.0 / _E) + beb_ref[...]
            summ = jax.nn.sigmoid(me)
            ws = lax.dot_general(summ, dw_ref[...], dims,
                                 preferred_element_type=jnp.float32)
            dims2 = (((1,), (0,)), ((), ()))
            u1 = lax.dot_general(ws, w1, dims2,
                                 preferred_element_type=jnp.float32)
            u2 = lax.dot_general(ws, w2, dims2,
                                 preferred_element_type=jnp.float32)
            u_s[...] = jnp.concatenate([u1, u2], axis=0)
            c_ref[...] = jnp.sum(beb_ref[...] * ws).reshape(1, 1)

    @pl.when(p == 1)
    def _phase1():
        u = u_s[...]
        hp = hp_s[pl.ds(off, _TN), :]
        hn = hn_s[pl.ds(off, _TN), :]
        dims = (((1,), (1,)), ((), ()))
        pq_p = lax.dot_general(u, hp, dims,
                               preferred_element_type=jnp.float32)
        pq_n = lax.dot_general(u, hn, dims,
                               preferred_element_type=jnp.float32)
        tab_ref[...] = jnp.concatenate(
            [pq_p, pq_n, jnp.zeros((4, _TN), jnp.float32)], axis=0)


def _run_b(nf, spos, sneg, din, dout, wnT, weT, brow, wew, beb_row, dw):
    grid = (2, _NP // _TN)

    def _i0(p, i):
        return jnp.where(p == 0, i, 0)

    seg = pl.BlockSpec((2, _TN, _EDIM), lambda p, i: (0, _i0(p, i), 0))
    return pl.pallas_call(
        _b_body,
        grid=grid,
        in_specs=[
            pl.BlockSpec((_TN, _DIN), lambda p, i: (_i0(p, i), 0)),
            seg, seg, seg, seg,
            pl.BlockSpec((_DIN, _H), lambda p, i: (0, 0)),
            pl.BlockSpec((_EDIM, _H), lambda p, i: (0, 0)),
            pl.BlockSpec((1, _H), lambda p, i: (0, 0)),
            pl.BlockSpec((_EOUT, _EOUT), lambda p, i: (0, 0)),
            pl.BlockSpec((1, _EOUT), lambda p, i: (0, 0)),
            pl.BlockSpec((_EOUT, _EOUT), lambda p, i: (0, 0)),
        ],
        out_specs=[pl.BlockSpec((8, _TN), lambda p, i: (p, i)),
                   pl.BlockSpec((1, 1), lambda p, i: (0, 0))],
        out_shape=[jax.ShapeDtypeStruct((16, _NP), jnp.float32),
                   jax.ShapeDtypeStruct((1, 1), jnp.float32)],
        scratch_shapes=[pltpu.VMEM((_NP, _H), jnp.float32),
                        pltpu.VMEM((_NP, _H), jnp.float32),
                        pltpu.VMEM((2, _H), jnp.float32),
                        pltpu.VMEM((2, _H), jnp.float32)],
        compiler_params=pltpu.CompilerParams(
            vmem_limit_bytes=100 * 1024 * 1024),
    )(nf, spos, sneg, din, dout, wnT, weT, brow, wew, beb_row, dw)


_LOG1P_C = (-6.07475245e-03, 3.44179115e-02, -9.23123095e-02,
            1.64781887e-01, -2.39189722e-01, 3.31333659e-01,
            -4.99801099e-01, 9.99991449e-01, 9.09903356e-08)


def _sc_edge_body(src, dst, tab, c_h, pp_o, pn_o,
                  tab_v, src_v, dst_v, c_v, part_v):
    cid = lax.axis_index("c")
    sid = lax.axis_index("s")
    wid = cid * _NS + sid

    pltpu.sync_copy(tab, tab_v)
    pltpu.sync_copy(c_h, c_v)
    base = wid * _EPW
    pltpu.sync_copy(src.at[pl.ds(base, _EPW)], src_v)
    pltpu.sync_copy(dst.at[pl.ds(base, _EPW)], dst_v)
    cv = c_v[...]

    def log1p_poly(t):
        p = jnp.full((16,), _LOG1P_C[0], jnp.float32)
        for coef in _LOG1P_C[1:]:
            p = p * t + coef
        return p

    def it(i, carry):
        ap, an = carry
        s = pl.ds(i * 16, 16)
        sv = src_v[s]
        dv = dst_v[s]
        xp = (plsc.load_gather(tab_v, [sv])
              + plsc.load_gather(tab_v, [dv + _NP]) + cv)
        xn = (plsc.load_gather(tab_v, [sv + 2 * _NP])
              + plsc.load_gather(tab_v, [dv + 3 * _NP]) + cv)
        ap = ap + jnp.maximum(-xp, 0.0) + log1p_poly(jnp.exp(-jnp.abs(xp)))
        an = an + jnp.maximum(xn, 0.0) + log1p_poly(jnp.exp(-jnp.abs(xn)))
        return (ap, an)

    z = jnp.zeros((16,), jnp.float32)
    ap, an = lax.fori_loop(0, _EPW // 16, it, (z, z))
    part_v[0, :] = ap
    part_v[1, :] = an
    out16 = pl.ds(wid * 16, 16)
    pltpu.sync_copy(part_v.at[0], pp_o.at[out16])
    pltpu.sync_copy(part_v.at[1], pn_o.at[out16])


def _run_sc_edge(src, dst, tab, c16):
    out = jax.ShapeDtypeStruct((_NW * 16,), jnp.float32)
    f = pl.kernel(
        _sc_edge_body,
        out_type=(out, out),
        mesh=_sc_mesh(),
        compiler_params=pltpu.CompilerParams(use_tc_tiling_on_sc=False,
                                             needs_layout_passes=False),
        scratch_types=[
            pltpu.VMEM((4 * _NP,), jnp.float32),
            pltpu.VMEM((_EPW,), jnp.int32),
            pltpu.VMEM((_EPW,), jnp.int32),
            pltpu.VMEM((16,), jnp.float32),
            pltpu.VMEM((2, 16), jnp.float32),
        ],
    )
    return f(src, dst, tab, c16)


def kernel(n_features, e_features, edge_index, W_apply_w, W_apply_b,
           W_edge_w, W_edge_b, disc_W):
    nf = n_features.reshape(_N, _DIN)
    ef_fm = e_features.reshape(_E, _EDIM).T
    src = edge_index[0]
    dst = edge_index[1]
    dstn = jnp.take(dst, jnp.asarray(_INV))

    spos, sneg, din, dout = _run_sc_scatter(ef_fm, src, dst, dstn)

    wnT = W_apply_w[:, :_DIN].T
    weT = W_apply_w[:, _DIN:].T
    brow = W_apply_b.reshape(1, _H)
    beb_row = W_edge_b.reshape(1, _EOUT)
    tab8, c = _run_b(nf, spos, sneg, din, dout,
                     wnT, weT, brow, W_edge_w, beb_row, disc_W)

    c16 = jnp.broadcast_to(c[0, 0], (16,))
    pp, pn = _run_sc_edge(src, dst, tab8[8:12].reshape(4 * _NP), c16)

    return (jnp.sum(pp) + jnp.sum(pn)) * (1.0 / _E)


# speedup vs baseline: 1.4177x; 1.0494x over previous
import numpy as np
import jax
import jax.numpy as jnp
from jax import lax
from jax.experimental import pallas as pl
from jax.experimental.pallas import tpu as pltpu
from jax.experimental.pallas import tpu_sc as plsc

_N = 10000
_E = 320000
_DIN = 128
_EDIM = 16
_H = 128
_EOUT = 256

_NC = 2
_NS = 16
_NW = _NC * _NS
_NP = 10240
_ROWS_PT = _NP // _NS
_EPW = _E // _NW
_BLK = 400
_NBLK = _EPW // _BLK
_TN = 640


def _make_perm():
    try:
        try:
            dev = jax.local_devices(backend="cpu")[0]
            with jax.default_device(dev):
                p = np.asarray(jax.random.permutation(jax.random.key(1), _E))
        except Exception:
            p = np.asarray(jax.random.permutation(jax.random.key(1), _E))
    except Exception:
        p = np.arange(_E)
    return p.astype(np.int32)


_PERM = _make_perm()
_INV = np.empty_like(_PERM)
_INV[_PERM] = np.arange(_E, dtype=np.int32)


def _sc_mesh():
    return plsc.VectorSubcoreMesh(core_axis_name="c", subcore_axis_name="s",
                                  num_cores=_NC, num_subcores=_NS)


def _sc_scatter_body(efeat_fm, src, dst, inv, zeros_h, ones_h,
                     spos_o, sneg_o, din_o, dout_o,
                     col0, col1, row0v, row1v, ones_v, zeros_v,
                     src0, src1, dst0, dst1, iv0, iv1, dn0, dn1,
                     acc_pos, acc_neg, acc_din, acc_dout, dst_sh,
                     seml0, seml1, semc0, semc1, semg0, semg1):
    cid = lax.axis_index("c")
    sid = lax.axis_index("s")
    wid = cid * _NS + sid

    pltpu.sync_copy(zeros_h, zeros_v)
    pltpu.sync_copy(ones_h, ones_v)

    stg = pl.ds(sid * (_E // _NS), _E // _NS)
    pltpu.sync_copy(dst.at[stg], dst_sh.at[stg])

    rr = sid * _ROWS_PT
    pltpu.sync_copy(zeros_v, acc_pos.at[pl.ds(rr, _ROWS_PT)])
    pltpu.sync_copy(zeros_v, acc_neg.at[pl.ds(rr, _ROWS_PT)])
    pltpu.sync_copy(zeros_v, acc_din.at[pl.ds(rr, _ROWS_PT)])
    pltpu.sync_copy(zeros_v, acc_dout.at[pl.ds(rr, _ROWS_PT)])
    plsc.subcore_barrier()

    slots = ((col0, row0v, src0, dst0, iv0, dn0, seml0, semc0, semg0),
             (col1, row1v, src1, dst1, iv1, dn1, seml1, semc1, semg1))
    lane = jax.lax.iota(jnp.int32, 16)

    def loads(b, s):
        cv, _, sv, dv, iv, _, seml, _, _ = slots[s]
        base = wid * _EPW + b * _BLK
        pltpu.async_copy(efeat_fm.at[:, pl.ds(base, _BLK)], cv, seml)
        pltpu.async_copy(src.at[pl.ds(base, _BLK)], sv, seml)
        pltpu.async_copy(dst.at[pl.ds(base, _BLK)], dv, seml)
        pltpu.async_copy(inv.at[pl.ds(base, _BLK)], iv, seml)

    def wait_loads(b, s):
        cv, _, sv, dv, iv, _, seml, _, _ = slots[s]
        base = wid * _EPW + b * _BLK
        pltpu.make_async_copy(efeat_fm.at[:, pl.ds(base, _BLK)], cv,
                              seml).wait()
        pltpu.make_async_copy(src.at[pl.ds(base, _BLK)], sv, seml).wait()
        pltpu.make_async_copy(dst.at[pl.ds(base, _BLK)], dv, seml).wait()
        pltpu.make_async_copy(inv.at[pl.ds(base, _BLK)], iv, seml).wait()

    def transpose(s):
        cv, rv = slots[s][0], slots[s][1]

        def tbody(j, carry):
            rows = lane + j * 16
            for k in range(_EDIM):
                v = cv[k, pl.ds(j * 16, 16)]
                plsc.store_scatter(
                    rv, [rows, jnp.full((16,), k, jnp.int32)], v)
            return carry

        lax.fori_loop(0, _BLK // 16, tbody, 0)

    def wait_scat(s):
        _, rv, sv, dv, _, nv, _, semc, _ = slots[s]
        pltpu.make_async_copy(rv, acc_pos.at[dv], semc).wait()
        pltpu.make_async_copy(rv, acc_neg.at[nv], semc).wait()
        pltpu.make_async_copy(ones_v, acc_din.at[dv], semc).wait()
        pltpu.make_async_copy(ones_v, acc_dout.at[sv], semc).wait()

    def half(b, s, t):
        _, rv, sv, dv, iv, nv, _, semc, semg = slots[s]
        wait_loads(b, s)
        g = pltpu.async_copy(dst_sh.at[iv], nv, semg)
        transpose(s)
        pltpu.async_copy(rv, acc_pos.at[dv], semc, add=True)
        pltpu.async_copy(ones_v, acc_din.at[dv], semc, add=True)
        pltpu.async_copy(ones_v, acc_dout.at[sv], semc, add=True)
        g.wait()
        pltpu.async_copy(rv, acc_neg.at[nv], semc, add=True)

        @pl.when(b >= 1)
        def _w():
            wait_scat(t)

        @pl.when(b + 1 < _NBLK)
        def _l():
            loads(b + 1, t)

    loads(0, 0)

    def g_body(g, carry):
        b0 = g * 2
        half(b0, 0, 1)

        @pl.when(b0 + 1 < _NBLK)
        def _h2():
            half(b0 + 1, 1, 0)

        return carry

    lax.fori_loop(0, (_NBLK + 1) // 2, g_body, 0)
    wait_scat((_NBLK - 1) % 2)
    plsc.subcore_barrier()

    rows = pl.ds(rr, _ROWS_PT)
    pltpu.sync_copy(acc_pos.at[rows], spos_o.at[cid, rows])
    pltpu.sync_copy(acc_neg.at[rows], sneg_o.at[cid, rows])
    pltpu.sync_copy(acc_din.at[rows], din_o.at[cid, rows])
    pltpu.sync_copy(acc_dout.at[rows], dout_o.at[cid, rows])


def _run_sc_scatter(ef_fm, src, dst, inv):
    acc = jax.ShapeDtypeStruct((_NC, _NP, _EDIM), jnp.float32)
    f = pl.kernel(
        _sc_scatter_body,
        out_type=(acc, acc, acc, acc),
        mesh=_sc_mesh(),
        compiler_params=pltpu.CompilerParams(use_tc_tiling_on_sc=False,
                                             needs_layout_passes=False),
        scratch_types=[
            pltpu.VMEM((_EDIM, _BLK), jnp.float32),
            pltpu.VMEM((_EDIM, _BLK), jnp.float32),
            pltpu.VMEM((_BLK, _EDIM), jnp.float32),
            pltpu.VMEM((_BLK, _EDIM), jnp.float32),
            pltpu.VMEM((_BLK, _EDIM), jnp.float32),
            pltpu.VMEM((_ROWS_PT, _EDIM), jnp.float32),
            pltpu.VMEM((_BLK,), jnp.int32),
            pltpu.VMEM((_BLK,), jnp.int32),
            pltpu.VMEM((_BLK,), jnp.int32),
            pltpu.VMEM((_BLK,), jnp.int32),
            pltpu.VMEM((_BLK,), jnp.int32),
            pltpu.VMEM((_BLK,), jnp.int32),
            pltpu.VMEM((_BLK,), jnp.int32),
            pltpu.VMEM((_BLK,), jnp.int32),
            pltpu.VMEM_SHARED((_NP, _EDIM), jnp.float32),
            pltpu.VMEM_SHARED((_NP, _EDIM), jnp.float32),
            pltpu.VMEM_SHARED((_NP, _EDIM), jnp.float32),
            pltpu.VMEM_SHARED((_NP, _EDIM), jnp.float32),
            pltpu.VMEM_SHARED((_E,), jnp.int32),
            pltpu.SemaphoreType.DMA,
            pltpu.SemaphoreType.DMA,
            pltpu.SemaphoreType.DMA,
            pltpu.SemaphoreType.DMA,
            pltpu.SemaphoreType.DMA,
            pltpu.SemaphoreType.DMA,
        ],
    )
    zeros_h = jnp.zeros((_ROWS_PT, _EDIM), jnp.float32)
    ones_h = jnp.ones((_BLK, _EDIM), jnp.float32)
    return f(ef_fm, src, dst, inv, zeros_h, ones_h)


def _b_body(nf_ref, spos_ref, sneg_ref, din_ref, dout_ref,
            wnT_ref, weT_ref, b_ref, wew_ref, beb_ref, dw_ref,
            tab_ref, c_ref, hp_s, hn_s, acc_ref, u_s):
    p = pl.program_id(0)
    i = pl.program_id(1)
    off = pl.multiple_of(i * _TN, _TN)

    @pl.when(p == 0)
    def _phase0():
        din = din_ref[0, :, 0:1] + din_ref[1, :, 0:1]
        dout = dout_ref[0, :, 0:1] + dout_ref[1, :, 0:1]
        inv = 1.0 / jnp.maximum(din, 1.0)
        sp = (spos_ref[0] + spos_ref[1]) * inv
        sn = (sneg_ref[0] + sneg_ref[1]) * inv
        base = jnp.dot(nf_ref[...], wnT_ref[...],
                       preferred_element_type=jnp.float32) + b_ref[...]
        weT = weT_ref[...]
        hp = jnp.maximum(
            base + jnp.dot(sp, weT, preferred_element_type=jnp.float32), 0.0)
        hn = jnp.maximum(
            base + jnp.dot(sn, weT, preferred_element_type=jnp.float32), 0.0)
        hp_s[pl.ds(off, _TN), :] = hp
        hn_s[pl.ds(off, _TN), :] = hn

        mask = ((lax.broadcasted_iota(jnp.int32, (_TN, 1), 0) + i * _TN)
                < _N).astype(jnp.float32)
        msrc_t = jnp.sum(hp * (dout * mask), axis=0, keepdims=True)
        mdst_t = jnp.sum(hp * (din * mask), axis=0, keepdims=True)

        @pl.when(i == 0)
        def _init():
            acc_ref[...] = jnp.zeros((2, _H), jnp.float32)

        acc_ref[0:1, :] += msrc_t
        acc_ref[1:2, :] += mdst_t

        @pl.when(i == (_NP // _TN) - 1)
        def _fin():
            msrc = acc_ref[0:1, :]
            mdst = acc_ref[1:2, :]
            wew = wew_ref[...]
            w1 = wew[:, :_H]
            w2 = wew[:, _H:]
            dims = (((1,), (1,)), ((), ()))
            me = (lax.dot_general(msrc, w1, dims,
                                  preferred_element_type=jnp.float32)
                  + lax.dot_general(mdst, w2, dims,
                                    preferred_element_type=jnp.float32)
                  ) * (1.0 / _E) + beb_ref[...]
            summ = jax.nn.sigmoid(me)
            ws = lax.dot_general(summ, dw_ref[...], dims,
                                 preferred_element_type=jnp.float32)
            dims2 = (((1,), (0,)), ((), ()))
            u1 = lax.dot_general(ws, w1, dims2,
                                 preferred_element_type=jnp.float32)
            u2 = lax.dot_general(ws, w2, dims2,
                                 preferred_element_type=jnp.float32)
            u_s[...] = jnp.concatenate([u1, u2], axis=0)
            c_ref[...] = jnp.sum(beb_ref[...] * ws).reshape(1, 1)

    @pl.when(p == 1)
    def _phase1():
        u = u_s[...]
        hp = hp_s[pl.ds(off, _TN), :]
        hn = hn_s[pl.ds(off, _TN), :]
        dims = (((1,), (1,)), ((), ()))
        pq_p = lax.dot_general(u, hp, dims,
                               preferred_element_type=jnp.float32)
        pq_n = lax.dot_general(u, hn, dims,
                               preferred_element_type=jnp.float32)
        tab_ref[...] = jnp.concatenate(
            [pq_p, pq_n, jnp.zeros((4, _TN), jnp.float32)], axis=0)


def _run_b(nf, spos, sneg, din, dout, wnT, weT, brow, wew, beb_row, dw):
    grid = (2, _NP // _TN)

    def _i0(p, i):
        return jnp.where(p == 0, i, 0)

    seg = pl.BlockSpec((2, _TN, _EDIM), lambda p, i: (0, _i0(p, i), 0))
    return pl.pallas_call(
        _b_body,
        grid=grid,
        in_specs=[
            pl.BlockSpec((_TN, _DIN), lambda p, i: (_i0(p, i), 0)),
            seg, seg, seg, seg,
            pl.BlockSpec((_DIN, _H), lambda p, i: (0, 0)),
            pl.BlockSpec((_EDIM, _H), lambda p, i: (0, 0)),
            pl.BlockSpec((1, _H), lambda p, i: (0, 0)),
            pl.BlockSpec((_EOUT, _EOUT), lambda p, i: (0, 0)),
            pl.BlockSpec((1, _EOUT), lambda p, i: (0, 0)),
            pl.BlockSpec((_EOUT, _EOUT), lambda p, i: (0, 0)),
        ],
        out_specs=[pl.BlockSpec((8, _TN), lambda p, i: (p, i)),
                   pl.BlockSpec((1, 1), lambda p, i: (0, 0))],
        out_shape=[jax.ShapeDtypeStruct((16, _NP), jnp.float32),
                   jax.ShapeDtypeStruct((1, 1), jnp.float32)],
        scratch_shapes=[pltpu.VMEM((_NP, _H), jnp.float32),
                        pltpu.VMEM((_NP, _H), jnp.float32),
                        pltpu.VMEM((2, _H), jnp.float32),
                        pltpu.VMEM((2, _H), jnp.float32)],
        compiler_params=pltpu.CompilerParams(
            vmem_limit_bytes=100 * 1024 * 1024),
    )(nf, spos, sneg, din, dout, wnT, weT, brow, wew, beb_row, dw)


_LOG1P_C = (-6.07475245e-03, 3.44179115e-02, -9.23123095e-02,
            1.64781887e-01, -2.39189722e-01, 3.31333659e-01,
            -4.99801099e-01, 9.99991449e-01, 9.09903356e-08)


def _sc_edge_body(src, dst, tab, c_h, pp_o, pn_o,
                  tab_v, src_v, dst_v, c_v, part_v):
    cid = lax.axis_index("c")
    sid = lax.axis_index("s")
    wid = cid * _NS + sid

    pltpu.sync_copy(tab, tab_v)
    pltpu.sync_copy(c_h, c_v)
    base = wid * _EPW
    pltpu.sync_copy(src.at[pl.ds(base, _EPW)], src_v)
    pltpu.sync_copy(dst.at[pl.ds(base, _EPW)], dst_v)
    cv = c_v[...]

    def log1p_poly(t):
        p = jnp.full((16,), _LOG1P_C[0], jnp.float32)
        for coef in _LOG1P_C[1:]:
            p = p * t + coef
        return p

    def it(i, carry):
        ap, an = carry
        s = pl.ds(i * 16, 16)
        sv = src_v[s]
        dv = dst_v[s]
        xp = (plsc.load_gather(tab_v, [sv])
              + plsc.load_gather(tab_v, [dv + _NP]) + cv)
        xn = (plsc.load_gather(tab_v, [sv + 2 * _NP])
              + plsc.load_gather(tab_v, [dv + 3 * _NP]) + cv)
        ap = ap + jnp.maximum(-xp, 0.0) + log1p_poly(jnp.exp(-jnp.abs(xp)))
        an = an + jnp.maximum(xn, 0.0) + log1p_poly(jnp.exp(-jnp.abs(xn)))
        return (ap, an)

    z = jnp.zeros((16,), jnp.float32)
    ap, an = lax.fori_loop(0, _EPW // 16, it, (z, z))
    part_v[0, :] = ap
    part_v[1, :] = an
    out16 = pl.ds(wid * 16, 16)
    pltpu.sync_copy(part_v.at[0], pp_o.at[out16])
    pltpu.sync_copy(part_v.at[1], pn_o.at[out16])


def _run_sc_edge(src, dst, tab, c16):
    out = jax.ShapeDtypeStruct((_NW * 16,), jnp.float32)
    f = pl.kernel(
        _sc_edge_body,
        out_type=(out, out),
        mesh=_sc_mesh(),
        compiler_params=pltpu.CompilerParams(use_tc_tiling_on_sc=False,
                                             needs_layout_passes=False),
        scratch_types=[
            pltpu.VMEM((4 * _NP,), jnp.float32),
            pltpu.VMEM((_EPW,), jnp.int32),
            pltpu.VMEM((_EPW,), jnp.int32),
            pltpu.VMEM((16,), jnp.float32),
            pltpu.VMEM((2, 16), jnp.float32),
        ],
    )
    return f(src, dst, tab, c16)


def kernel(n_features, e_features, edge_index, W_apply_w, W_apply_b,
           W_edge_w, W_edge_b, disc_W):
    nf = n_features.reshape(_N, _DIN)
    ef_fm = e_features.reshape(_E, _EDIM).T
    src = edge_index[0]
    dst = edge_index[1]
    spos, sneg, din, dout = _run_sc_scatter(ef_fm, src, dst,
                                            jnp.asarray(_INV))

    wnT = W_apply_w[:, :_DIN].T
    weT = W_apply_w[:, _DIN:].T
    brow = W_apply_b.reshape(1, _H)
    beb_row = W_edge_b.reshape(1, _EOUT)
    tab8, c = _run_b(nf, spos, sneg, din, dout,
                     wnT, weT, brow, W_edge_w, beb_row, disc_W)

    c16 = jnp.broadcast_to(c[0, 0], (16,))
    pp, pn = _run_sc_edge(src, dst, tab8[8:12].reshape(4 * _NP), c16)

    return (jnp.sum(pp) + jnp.sum(pn)) * (1.0 / _E)
